# baseline (speedup 1.0000x reference)
import functools

import jax
import jax.numpy as jnp
from jax.experimental import pallas as pl
from jax.experimental.pallas import tpu as pltpu

B, N, CFEAT = 16, 4096, 64
NPOINT, NSAMPLE = 1024, 32
EPS = 1e-5


def _gather_rows(points, idx):
    return jax.vmap(lambda p, i: p[i])(points, idx)


def _fps(xyz, npoint):
    b, n, _ = xyz.shape
    def body(i, state):
        centroids, distance, farthest = state
        centroids = centroids.at[:, i].set(farthest)
        centroid = jax.vmap(lambda p, f: p[f])(xyz, farthest)[:, None, :]
        dist = jnp.sum((xyz - centroid) ** 2, -1)
        distance = jnp.minimum(distance, dist)
        farthest = jnp.argmax(distance, -1).astype(jnp.int32)
        return centroids, distance, farthest
    init = (jnp.zeros((b, npoint), jnp.int32),
            jnp.full((b, n), 1e10, jnp.float32),
            jnp.zeros((b,), jnp.int32))
    centroids, _, _ = jax.lax.fori_loop(0, npoint, body, init)
    return centroids


def _knn(xyz, query, k):
    dist = -2.0 * jnp.einsum('bmc,bnc->bmn', query, xyz)
    dist = dist + jnp.sum(query ** 2, -1)[:, :, None]
    dist = dist + jnp.sum(xyz ** 2, -1)[:, None, :]
    _, idx = jax.lax.top_k(-dist, k)
    return idx


def _mlp_bn_max(g, params):
    for W, bb, gamma, beta in params:
        g = jnp.einsum('bmkc,oc->bmko', g, W) + bb
        mean = jnp.mean(g, axis=(0, 1, 2), keepdims=True)
        var = jnp.var(g, axis=(0, 1, 2), keepdims=True)
        g = (g - mean) / jnp.sqrt(var + EPS) * gamma + beta
        g = jax.nn.relu(g)
    return jnp.max(g, axis=2)


def _identity_kernel(x_ref, o_ref):
    o_ref[...] = x_ref[...]


def _pallas_identity(x):
    return pl.pallas_call(
        _identity_kernel,
        out_shape=jax.ShapeDtypeStruct(x.shape, x.dtype),
    )(x)


def kernel(xyz, points, W0, b0, gamma0, beta0, W1, b1, gamma1, beta1, W2, b2, gamma2, beta2):
    fps_idx = _fps(xyz, NPOINT)
    new_xyz = _gather_rows(xyz, fps_idx)
    idx = _knn(xyz, new_xyz, NSAMPLE)
    grouped_xyz = _gather_rows(xyz, idx) - new_xyz[:, :, None, :]
    grouped_points = jnp.concatenate([grouped_xyz, _gather_rows(points, idx)], axis=-1)
    new_points = _mlp_bn_max(grouped_points,
                             [(W0, b0, gamma0, beta0), (W1, b1, gamma1, beta1), (W2, b2, gamma2, beta2)])
    return _pallas_identity(new_xyz), new_points


# speedup vs baseline: 1.5001x; 1.5001x over previous
import functools

import jax
import jax.numpy as jnp
from jax.experimental import pallas as pl
from jax.experimental.pallas import tpu as pltpu

B, N, CFEAT = 16, 4096, 64
NPOINT, NSAMPLE = 1024, 32
EPS = 1e-5


def _gather_rows(points, idx):
    return jax.vmap(lambda p, i: p[i])(points, idx)


_ROWS = N // 8


def _coset_reduce(v, op):
    for s in (16, 32, 64):
        v = op(v, pltpu.roll(v, s, 1))
    return v


def _fps_body(xt_ref, narr_ref, ox_ref, oy_ref, oz_ref):
    x = xt_ref[0]
    y = xt_ref[1]
    z = xt_ref[2]
    narr = narr_ref[...]
    rowi = jax.lax.broadcasted_iota(jnp.int32, (8, 128), 0)

    def step(i, carry):
        distance, far, ax, ay, az = carry
        sel = (narr == far).astype(jnp.float32)
        cx = _coset_reduce(jnp.sum(x * sel, axis=0, keepdims=True), jnp.add)
        cy = _coset_reduce(jnp.sum(y * sel, axis=0, keepdims=True), jnp.add)
        cz = _coset_reduce(jnp.sum(z * sel, axis=0, keepdims=True), jnp.add)

        r = i & 7
        keep = (r != 0).astype(jnp.float32)
        maskr = (rowi == r).astype(jnp.float32)
        ax = ax * keep + cx * maskr
        ay = ay * keep + cy * maskr
        az = az * keep + cz * maskr

        @pl.when(r == 7)
        def _flush():
            base = pl.multiple_of(i - 7, 8)
            ox_ref[pl.ds(base, 8), :] = ax
            oy_ref[pl.ds(base, 8), :] = ay
            oz_ref[pl.ds(base, 8), :] = az

        dx = x - cx
        dy = y - cy
        dz = z - cz
        dist = (dx * dx + dy * dy) + dz * dz
        distance = jnp.minimum(distance, dist)
        m = _coset_reduce(jnp.max(distance, axis=0, keepdims=True), jnp.maximum)
        eq = (distance == m).astype(jnp.float32)
        cand = narr * eq + N * (1.0 - eq)
        far = _coset_reduce(jnp.min(cand, axis=0, keepdims=True), jnp.minimum)
        return distance, far, ax, ay, az

    zero8 = jnp.zeros((8, 128), jnp.float32)
    init = (jnp.full((_ROWS, 128), 1e10, jnp.float32),
            jnp.zeros((1, 128), jnp.float32), zero8, zero8, zero8)
    jax.lax.fori_loop(0, NPOINT, step, init)


def _fps_pallas(xyz):
    xt = jnp.transpose(xyz.reshape(B, _ROWS, 8, 3), (3, 1, 2, 0)).reshape(3, _ROWS, 128)
    n_ids = jnp.arange(N, dtype=jnp.float32).reshape(_ROWS, 8)
    narr = jnp.broadcast_to(n_ids[:, :, None], (_ROWS, 8, B)).reshape(_ROWS, 128)
    ox, oy, oz = pl.pallas_call(
        _fps_body,
        out_shape=(jax.ShapeDtypeStruct((NPOINT, 128), jnp.float32),) * 3,
    )(xt, narr)
    new_xyz = jnp.stack([ox[:, :B].T, oy[:, :B].T, oz[:, :B].T], axis=-1)
    return new_xyz


def _knn(xyz, query, k):
    dist = -2.0 * jnp.einsum('bmc,bnc->bmn', query, xyz)
    dist = dist + jnp.sum(query ** 2, -1)[:, :, None]
    dist = dist + jnp.sum(xyz ** 2, -1)[:, None, :]
    _, idx = jax.lax.top_k(-dist, k)
    return idx


def _mlp_bn_max(g, params):
    for W, bb, gamma, beta in params:
        g = jnp.einsum('bmkc,oc->bmko', g, W) + bb
        mean = jnp.mean(g, axis=(0, 1, 2), keepdims=True)
        var = jnp.var(g, axis=(0, 1, 2), keepdims=True)
        g = (g - mean) / jnp.sqrt(var + EPS) * gamma + beta
        g = jax.nn.relu(g)
    return jnp.max(g, axis=2)


def _identity_kernel(x_ref, o_ref):
    o_ref[...] = x_ref[...]


def _pallas_identity(x):
    return pl.pallas_call(
        _identity_kernel,
        out_shape=jax.ShapeDtypeStruct(x.shape, x.dtype),
    )(x)


def kernel(xyz, points, W0, b0, gamma0, beta0, W1, b1, gamma1, beta1, W2, b2, gamma2, beta2):
    new_xyz = _fps_pallas(xyz)
    idx = _knn(xyz, new_xyz, NSAMPLE)
    grouped_xyz = _gather_rows(xyz, idx) - new_xyz[:, :, None, :]
    grouped_points = jnp.concatenate([grouped_xyz, _gather_rows(points, idx)], axis=-1)
    new_points = _mlp_bn_max(grouped_points,
                             [(W0, b0, gamma0, beta0), (W1, b1, gamma1, beta1), (W2, b2, gamma2, beta2)])
    return new_xyz, new_points


# speedup vs baseline: 2.2811x; 1.5207x over previous
import functools

import jax
import jax.numpy as jnp
from jax.experimental import pallas as pl
from jax.experimental.pallas import tpu as pltpu

B, N, CFEAT = 16, 4096, 64
NPOINT, NSAMPLE = 1024, 32
EPS = 1e-5


def _gather_rows(points, idx):
    return jax.vmap(lambda p, i: p[i])(points, idx)


_ROWS = N // 8


def _coset_reduce(v, op):
    for s in (16, 32, 64):
        v = op(v, pltpu.roll(v, s, 1))
    return v


def _fps_body(xt_ref, narr_ref, ox_ref, oy_ref, oz_ref):
    x = xt_ref[0]
    y = xt_ref[1]
    z = xt_ref[2]
    narr = narr_ref[...]
    rowi = jax.lax.broadcasted_iota(jnp.int32, (8, 128), 0)

    def step(i, carry):
        distance, far, ax, ay, az = carry
        sel = (narr == far).astype(jnp.float32)
        cx = _coset_reduce(jnp.sum(x * sel, axis=0, keepdims=True), jnp.add)
        cy = _coset_reduce(jnp.sum(y * sel, axis=0, keepdims=True), jnp.add)
        cz = _coset_reduce(jnp.sum(z * sel, axis=0, keepdims=True), jnp.add)

        r = i & 7
        keep = (r != 0).astype(jnp.float32)
        maskr = (rowi == r).astype(jnp.float32)
        ax = ax * keep + cx * maskr
        ay = ay * keep + cy * maskr
        az = az * keep + cz * maskr

        @pl.when(r == 7)
        def _flush():
            base = pl.multiple_of(i - 7, 8)
            ox_ref[pl.ds(base, 8), :] = ax
            oy_ref[pl.ds(base, 8), :] = ay
            oz_ref[pl.ds(base, 8), :] = az

        dx = x - cx
        dy = y - cy
        dz = z - cz
        dist = (dx * dx + dy * dy) + dz * dz
        distance = jnp.minimum(distance, dist)
        m = _coset_reduce(jnp.max(distance, axis=0, keepdims=True), jnp.maximum)
        eq = (distance == m).astype(jnp.float32)
        cand = narr * eq + N * (1.0 - eq)
        far = _coset_reduce(jnp.min(cand, axis=0, keepdims=True), jnp.minimum)
        return distance, far, ax, ay, az

    zero8 = jnp.zeros((8, 128), jnp.float32)
    init = (jnp.full((_ROWS, 128), 1e10, jnp.float32),
            jnp.zeros((1, 128), jnp.float32), zero8, zero8, zero8)
    jax.lax.fori_loop(0, NPOINT, step, init)


def _fps_pallas(xyz):
    xt = jnp.transpose(xyz.reshape(B, _ROWS, 8, 3), (3, 1, 2, 0)).reshape(3, _ROWS, 128)
    n_ids = jnp.arange(N, dtype=jnp.float32).reshape(_ROWS, 8)
    narr = jnp.broadcast_to(n_ids[:, :, None], (_ROWS, 8, B)).reshape(_ROWS, 128)
    ox, oy, oz = pl.pallas_call(
        _fps_body,
        out_shape=(jax.ShapeDtypeStruct((NPOINT, 128), jnp.float32),) * 3,
    )(xt, narr)
    new_xyz = jnp.stack([ox[:, :B].T, oy[:, :B].T, oz[:, :B].T], axis=-1)
    return new_xyz


_MBLK = 256
_NG = N // 128
_BIGV = 3.0e38
_BIGN = 1.0e9


def _knn_body(q_ref, xx_ref, xy_ref, xz_ref, n128_ref, out_ref, dist_ref):
    qx = q_ref[0, :, 0:1]
    qy = q_ref[0, :, 1:2]
    qz = q_ref[0, :, 2:3]
    xx = xx_ref[0]
    xy = xy_ref[0]
    xz = xz_ref[0]
    n128 = n128_ref[...]

    def _b(v):
        return v.astype(jnp.bfloat16).astype(jnp.float32)

    e = (_b(qx) * _b(xx) + _b(qy) * _b(xy)) + _b(qz) * _b(xz)
    q2 = (qx * qx + qy * qy) + qz * qz
    x2 = (xx * xx + xy * xy) + xz * xz
    dist_ref[...] = (-2.0 * e + q2) + x2

    def lexmin(v, n, axis):
        m = jnp.min(v, axis=axis, keepdims=True)
        eq = (v == m).astype(jnp.float32)
        nm = jnp.min(n * eq + _BIGN * (1.0 - eq), axis=axis, keepdims=True)
        return m, eq, nm

    def round_body(carry):
        done, best_v, best_n, r = carry
        L = dist_ref[:, 0:128]
        for g in range(1, _NG):
            L = jnp.minimum(L, dist_ref[:, g * 128:(g + 1) * 128])
        nmin = jnp.full((_MBLK, 128), _BIGN, jnp.float32)
        for g in range(_NG):
            dg = dist_ref[:, g * 128:(g + 1) * 128]
            eq = (dg == L).astype(jnp.float32)
            ng = n128 + jnp.float32(g * 128)
            nmin = jnp.minimum(nmin, ng * eq + _BIGN * (1.0 - eq))
        rv, _, rn = lexmin(L, nmin, 1)
        wv = jnp.max(best_v, axis=1, keepdims=True)
        eqw = (best_v == wv).astype(jnp.float32)
        wn = jnp.max(best_n * eqw - (1.0 - eqw), axis=1, keepdims=True)
        row_done = (rv > wv) | ((rv == wv) & (rn > wn))
        new_done = jnp.all(row_done)
        for g in range(_NG):
            dg = dist_ref[:, g * 128:(g + 1) * 128]
            ng = n128 + jnp.float32(g * 128)
            kill = ((dg == L) & (ng == nmin)).astype(jnp.float32)
            dist_ref[:, g * 128:(g + 1) * 128] = dg * (1.0 - kill) + _BIGV * kill
        cv = jnp.concatenate([best_v, L], axis=1)
        cn = jnp.concatenate([best_n, nmin], axis=1)
        bvs, bns = [], []
        for _ in range(NSAMPLE):
            m, eq, nsel = lexmin(cv, cn, 1)
            bvs.append(m)
            bns.append(nsel)
            kill = eq * (cn == nsel).astype(jnp.float32)
            cv = cv * (1.0 - kill) + _BIGV * kill
        best_v = jnp.concatenate(bvs, axis=1)
        best_n = jnp.concatenate(bns, axis=1)
        return new_done, best_v, best_n, r + 1

    def cond(carry):
        done, _, _, r = carry
        return jnp.logical_not(done) & (r < 40)

    init = (jnp.bool_(False),
            jnp.full((_MBLK, NSAMPLE), _BIGV, jnp.float32),
            jnp.full((_MBLK, NSAMPLE), _BIGN, jnp.float32),
            jnp.int32(0))
    _, _, best_n, _ = jax.lax.while_loop(cond, round_body, init)
    out_ref[0] = best_n.astype(jnp.int32)


def _knn_pallas(xyz, new_xyz):
    xx = xyz[:, :, 0].reshape(B, 1, N)
    xy = xyz[:, :, 1].reshape(B, 1, N)
    xz = xyz[:, :, 2].reshape(B, 1, N)
    n128 = jnp.arange(128, dtype=jnp.float32).reshape(1, 128)
    grid = (B, NPOINT // _MBLK)
    idx = pl.pallas_call(
        _knn_body,
        grid=grid,
        in_specs=[
            pl.BlockSpec((1, _MBLK, 3), lambda b, m: (b, m, 0)),
            pl.BlockSpec((1, 1, N), lambda b, m: (b, 0, 0)),
            pl.BlockSpec((1, 1, N), lambda b, m: (b, 0, 0)),
            pl.BlockSpec((1, 1, N), lambda b, m: (b, 0, 0)),
            pl.BlockSpec((1, 128), lambda b, m: (0, 0)),
        ],
        out_specs=pl.BlockSpec((1, _MBLK, NSAMPLE), lambda b, m: (b, m, 0)),
        out_shape=jax.ShapeDtypeStruct((B, NPOINT, NSAMPLE), jnp.int32),
        scratch_shapes=[pltpu.VMEM((_MBLK, N), jnp.float32)],
    )(new_xyz, xx, xy, xz, n128)
    return idx


def _mlp_bn_max(g, params):
    for W, bb, gamma, beta in params:
        g = jnp.einsum('bmkc,oc->bmko', g, W) + bb
        mean = jnp.mean(g, axis=(0, 1, 2), keepdims=True)
        var = jnp.var(g, axis=(0, 1, 2), keepdims=True)
        g = (g - mean) / jnp.sqrt(var + EPS) * gamma + beta
        g = jax.nn.relu(g)
    return jnp.max(g, axis=2)


def _identity_kernel(x_ref, o_ref):
    o_ref[...] = x_ref[...]


def _pallas_identity(x):
    return pl.pallas_call(
        _identity_kernel,
        out_shape=jax.ShapeDtypeStruct(x.shape, x.dtype),
    )(x)


def kernel(xyz, points, W0, b0, gamma0, beta0, W1, b1, gamma1, beta1, W2, b2, gamma2, beta2):
    new_xyz = _fps_pallas(xyz)
    idx = _knn_pallas(xyz, new_xyz)
    grouped_xyz = _gather_rows(xyz, idx) - new_xyz[:, :, None, :]
    grouped_points = jnp.concatenate([grouped_xyz, _gather_rows(points, idx)], axis=-1)
    new_points = _mlp_bn_max(grouped_points,
                             [(W0, b0, gamma0, beta0), (W1, b1, gamma1, beta1), (W2, b2, gamma2, beta2)])
    return new_xyz, new_points


# speedup vs baseline: 2.3252x; 1.0193x over previous
import functools

import jax
import jax.numpy as jnp
from jax.experimental import pallas as pl
from jax.experimental.pallas import tpu as pltpu

B, N, CFEAT = 16, 4096, 64
NPOINT, NSAMPLE = 1024, 32
MLP = [64, 64, 128]
EPS = 1e-5


def _gather_rows(points, idx):
    return jax.vmap(lambda p, i: p[i])(points, idx)


_ROWS = N // 8


def _coset_reduce(v, op):
    for s in (16, 32, 64):
        v = op(v, pltpu.roll(v, s, 1))
    return v


def _fps_body(xt_ref, narr_ref, ox_ref, oy_ref, oz_ref):
    x = xt_ref[0]
    y = xt_ref[1]
    z = xt_ref[2]
    narr = narr_ref[...]
    rowi = jax.lax.broadcasted_iota(jnp.int32, (8, 128), 0)

    def step(i, carry):
        distance, far, ax, ay, az = carry
        sel = (narr == far).astype(jnp.float32)
        cx = _coset_reduce(jnp.sum(x * sel, axis=0, keepdims=True), jnp.add)
        cy = _coset_reduce(jnp.sum(y * sel, axis=0, keepdims=True), jnp.add)
        cz = _coset_reduce(jnp.sum(z * sel, axis=0, keepdims=True), jnp.add)

        r = i & 7
        keep = (r != 0).astype(jnp.float32)
        maskr = (rowi == r).astype(jnp.float32)
        ax = ax * keep + cx * maskr
        ay = ay * keep + cy * maskr
        az = az * keep + cz * maskr

        @pl.when(r == 7)
        def _flush():
            base = pl.multiple_of(i - 7, 8)
            ox_ref[pl.ds(base, 8), :] = ax
            oy_ref[pl.ds(base, 8), :] = ay
            oz_ref[pl.ds(base, 8), :] = az

        dx = x - cx
        dy = y - cy
        dz = z - cz
        dist = (dx * dx + dy * dy) + dz * dz
        distance = jnp.minimum(distance, dist)
        m = _coset_reduce(jnp.max(distance, axis=0, keepdims=True), jnp.maximum)
        eq = (distance == m).astype(jnp.float32)
        cand = narr * eq + N * (1.0 - eq)
        far = _coset_reduce(jnp.min(cand, axis=0, keepdims=True), jnp.minimum)
        return distance, far, ax, ay, az

    zero8 = jnp.zeros((8, 128), jnp.float32)
    init = (jnp.full((_ROWS, 128), 1e10, jnp.float32),
            jnp.zeros((1, 128), jnp.float32), zero8, zero8, zero8)
    jax.lax.fori_loop(0, NPOINT, step, init)


def _fps_pallas(xyz):
    xt = jnp.transpose(xyz.reshape(B, _ROWS, 8, 3), (3, 1, 2, 0)).reshape(3, _ROWS, 128)
    n_ids = jnp.arange(N, dtype=jnp.float32).reshape(_ROWS, 8)
    narr = jnp.broadcast_to(n_ids[:, :, None], (_ROWS, 8, B)).reshape(_ROWS, 128)
    ox, oy, oz = pl.pallas_call(
        _fps_body,
        out_shape=(jax.ShapeDtypeStruct((NPOINT, 128), jnp.float32),) * 3,
    )(xt, narr)
    new_xyz = jnp.stack([ox[:, :B].T, oy[:, :B].T, oz[:, :B].T], axis=-1)
    return new_xyz


_MBLK = 256
_NG = N // 128
_BIGV = 3.0e38
_BIGN = 1.0e9


def _knn_body(q_ref, xx_ref, xy_ref, xz_ref, n128_ref, out_ref, dist_ref):
    qx = q_ref[0, :, 0:1]
    qy = q_ref[0, :, 1:2]
    qz = q_ref[0, :, 2:3]
    xx = xx_ref[0]
    xy = xy_ref[0]
    xz = xz_ref[0]
    n128 = n128_ref[...]

    def _b(v):
        return v.astype(jnp.bfloat16).astype(jnp.float32)

    e = (_b(qx) * _b(xx) + _b(qy) * _b(xy)) + _b(qz) * _b(xz)
    q2 = (qx * qx + qy * qy) + qz * qz
    x2 = (xx * xx + xy * xy) + xz * xz
    dist_ref[...] = (-2.0 * e + q2) + x2

    def lexmin(v, n, axis):
        m = jnp.min(v, axis=axis, keepdims=True)
        eq = (v == m).astype(jnp.float32)
        nm = jnp.min(n * eq + _BIGN * (1.0 - eq), axis=axis, keepdims=True)
        return m, eq, nm

    def round_body(carry):
        done, best_v, best_n, r = carry
        L = dist_ref[:, 0:128]
        for g in range(1, _NG):
            L = jnp.minimum(L, dist_ref[:, g * 128:(g + 1) * 128])
        nmin = jnp.full((_MBLK, 128), _BIGN, jnp.float32)
        for g in range(_NG):
            dg = dist_ref[:, g * 128:(g + 1) * 128]
            eq = (dg == L).astype(jnp.float32)
            ng = n128 + jnp.float32(g * 128)
            nmin = jnp.minimum(nmin, ng * eq + _BIGN * (1.0 - eq))
        rv, _, rn = lexmin(L, nmin, 1)
        wv = jnp.max(best_v, axis=1, keepdims=True)
        eqw = (best_v == wv).astype(jnp.float32)
        wn = jnp.max(best_n * eqw - (1.0 - eqw), axis=1, keepdims=True)
        row_done = (rv > wv) | ((rv == wv) & (rn > wn))
        new_done = jnp.all(row_done)
        for g in range(_NG):
            dg = dist_ref[:, g * 128:(g + 1) * 128]
            ng = n128 + jnp.float32(g * 128)
            kill = ((dg == L) & (ng == nmin)).astype(jnp.float32)
            dist_ref[:, g * 128:(g + 1) * 128] = dg * (1.0 - kill) + _BIGV * kill
        cv = jnp.concatenate([best_v, L], axis=1)
        cn = jnp.concatenate([best_n, nmin], axis=1)
        bvs, bns = [], []
        for _ in range(NSAMPLE):
            m, eq, nsel = lexmin(cv, cn, 1)
            bvs.append(m)
            bns.append(nsel)
            kill = eq * (cn == nsel).astype(jnp.float32)
            cv = cv * (1.0 - kill) + _BIGV * kill
        best_v = jnp.concatenate(bvs, axis=1)
        best_n = jnp.concatenate(bns, axis=1)
        return new_done, best_v, best_n, r + 1

    def cond(carry):
        done, _, _, r = carry
        return jnp.logical_not(done) & (r < 40)

    init = (jnp.bool_(False),
            jnp.full((_MBLK, NSAMPLE), _BIGV, jnp.float32),
            jnp.full((_MBLK, NSAMPLE), _BIGN, jnp.float32),
            jnp.int32(0))
    _, _, best_n, _ = jax.lax.while_loop(cond, round_body, init)
    out_ref[0] = best_n.astype(jnp.int32)


def _knn_pallas(xyz, new_xyz):
    xx = xyz[:, :, 0].reshape(B, 1, N)
    xy = xyz[:, :, 1].reshape(B, 1, N)
    xz = xyz[:, :, 2].reshape(B, 1, N)
    n128 = jnp.arange(128, dtype=jnp.float32).reshape(1, 128)
    grid = (B, NPOINT // _MBLK)
    idx = pl.pallas_call(
        _knn_body,
        grid=grid,
        in_specs=[
            pl.BlockSpec((1, _MBLK, 3), lambda b, m: (b, m, 0)),
            pl.BlockSpec((1, 1, N), lambda b, m: (b, 0, 0)),
            pl.BlockSpec((1, 1, N), lambda b, m: (b, 0, 0)),
            pl.BlockSpec((1, 1, N), lambda b, m: (b, 0, 0)),
            pl.BlockSpec((1, 128), lambda b, m: (0, 0)),
        ],
        out_specs=pl.BlockSpec((1, _MBLK, NSAMPLE), lambda b, m: (b, m, 0)),
        out_shape=jax.ShapeDtypeStruct((B, NPOINT, NSAMPLE), jnp.int32),
        scratch_shapes=[pltpu.VMEM((_MBLK, N), jnp.float32)],
    )(new_xyz, xx, xy, xz, n128)
    return idx


_P_ROWS = B * NPOINT * NSAMPLE
_RBLK = 4096
_GRID_MLP = _P_ROWS // _RBLK


def _mm_bias(g, w, bias_row):
    return jnp.dot(g.astype(jnp.bfloat16), w.astype(jnp.bfloat16),
                   preferred_element_type=jnp.float32) + bias_row


def _acc_stats(st_ref, pre):
    s = jnp.sum(pre, axis=0, keepdims=True)
    s2 = jnp.sum(pre * pre, axis=0, keepdims=True)

    @pl.when(pl.program_id(0) == 0)
    def _():
        st_ref[...] = jnp.zeros_like(st_ref)

    st_ref[0:1, :] = st_ref[0:1, :] + s
    st_ref[1:2, :] = st_ref[1:2, :] + s2


def _norm_relu(pre, st, gamma, beta):
    mean = st[0:1, :] * (1.0 / _P_ROWS)
    var = st[1:2, :] * (1.0 / _P_ROWS) - mean * mean
    rs = 1.0 / jnp.sqrt(var + EPS)
    return jnp.maximum((pre - mean) * rs * gamma + beta, 0.0)


def _pass1_body(g_ref, w_ref, b_ref, pre_ref, st_ref):
    pre = _mm_bias(g_ref[...], w_ref[...], b_ref[...])
    pre_ref[...] = pre
    _acc_stats(st_ref, pre)


def _pass2_body(pre_ref, stin_ref, gam_ref, bet_ref, w_ref, b_ref, out_ref, st_ref):
    h = _norm_relu(pre_ref[...], stin_ref[...], gam_ref[...], bet_ref[...])
    pre2 = _mm_bias(h, w_ref[...], b_ref[...])
    out_ref[...] = pre2
    _acc_stats(st_ref, pre2)


def _pass3_body(pre_ref, stin_ref, gam_ref, bet_ref, w_ref, b_ref,
                mx_ref, mn_ref, st_ref):
    h = _norm_relu(pre_ref[...], stin_ref[...], gam_ref[...], bet_ref[...])
    pre3 = _mm_bias(h, w_ref[...], b_ref[...])
    _acc_stats(st_ref, pre3)
    r = pre3.reshape(_RBLK // NSAMPLE, NSAMPLE, MLP[2])
    mx_ref[...] = jnp.max(r, axis=1)
    mn_ref[...] = jnp.min(r, axis=1)


def _pass4_body(mx_ref, mn_ref, st_ref, gam_ref, bet_ref, out_ref):
    st = st_ref[...]
    mean = st[0:1, :] * (1.0 / _P_ROWS)
    var = st[1:2, :] * (1.0 / _P_ROWS) - mean * mean
    rs = 1.0 / jnp.sqrt(var + EPS)
    gamma = gam_ref[...]
    gpos = (gamma > 0.0).astype(jnp.float32)
    sel = mx_ref[...] * gpos + mn_ref[...] * (1.0 - gpos)
    out_ref[...] = jnp.maximum((sel - mean) * rs * gamma + bet_ref[...], 0.0)


def _row_spec(blk, cols):
    return pl.BlockSpec((blk, cols), lambda i: (i, 0))


def _const_spec(rows, cols):
    return pl.BlockSpec((rows, cols), lambda i: (0, 0))


def _mlp_bn_max_pallas(grouped, params):
    (W0, b0, g0, be0), (W1, b1, g1, be1), (W2, b2, g2, be2) = params
    c_in = grouped.shape[-1]
    rows = grouped.reshape(_P_ROWS, c_in)
    row1 = lambda v: v.reshape(1, -1)

    pre1, st1 = pl.pallas_call(
        _pass1_body, grid=(_GRID_MLP,),
        in_specs=[_row_spec(_RBLK, c_in), _const_spec(c_in, MLP[0]), _const_spec(1, MLP[0])],
        out_specs=(_row_spec(_RBLK, MLP[0]), _const_spec(2, MLP[0])),
        out_shape=(jax.ShapeDtypeStruct((_P_ROWS, MLP[0]), jnp.float32),
                   jax.ShapeDtypeStruct((2, MLP[0]), jnp.float32)),
    )(rows, W0.T, row1(b0))

    pre2, st2 = pl.pallas_call(
        _pass2_body, grid=(_GRID_MLP,),
        in_specs=[_row_spec(_RBLK, MLP[0]), _const_spec(2, MLP[0]),
                  _const_spec(1, MLP[0]), _const_spec(1, MLP[0]),
                  _const_spec(MLP[0], MLP[1]), _const_spec(1, MLP[1])],
        out_specs=(_row_spec(_RBLK, MLP[1]), _const_spec(2, MLP[1])),
        out_shape=(jax.ShapeDtypeStruct((_P_ROWS, MLP[1]), jnp.float32),
                   jax.ShapeDtypeStruct((2, MLP[1]), jnp.float32)),
    )(pre1, st1, row1(g0), row1(be0), W1.T, row1(b1))

    nc = _P_ROWS // NSAMPLE
    cblk = _RBLK // NSAMPLE
    mx, mn, st3 = pl.pallas_call(
        _pass3_body, grid=(_GRID_MLP,),
        in_specs=[_row_spec(_RBLK, MLP[1]), _const_spec(2, MLP[1]),
                  _const_spec(1, MLP[1]), _const_spec(1, MLP[1]),
                  _const_spec(MLP[1], MLP[2]), _const_spec(1, MLP[2])],
        out_specs=(_row_spec(cblk, MLP[2]), _row_spec(cblk, MLP[2]),
                   _const_spec(2, MLP[2])),
        out_shape=(jax.ShapeDtypeStruct((nc, MLP[2]), jnp.float32),
                   jax.ShapeDtypeStruct((nc, MLP[2]), jnp.float32),
                   jax.ShapeDtypeStruct((2, MLP[2]), jnp.float32)),
    )(pre2, st2, row1(g1), row1(be1), W2.T, row1(b2))

    out = pl.pallas_call(
        _pass4_body, grid=(8,),
        in_specs=[_row_spec(nc // 8, MLP[2]), _row_spec(nc // 8, MLP[2]),
                  _const_spec(2, MLP[2]), _const_spec(1, MLP[2]), _const_spec(1, MLP[2])],
        out_specs=_row_spec(nc // 8, MLP[2]),
        out_shape=jax.ShapeDtypeStruct((nc, MLP[2]), jnp.float32),
    )(mx, mn, st3, row1(g2), row1(be2))
    return out.reshape(B, NPOINT, MLP[2])


def _identity_kernel(x_ref, o_ref):
    o_ref[...] = x_ref[...]


def _pallas_identity(x):
    return pl.pallas_call(
        _identity_kernel,
        out_shape=jax.ShapeDtypeStruct(x.shape, x.dtype),
    )(x)


def kernel(xyz, points, W0, b0, gamma0, beta0, W1, b1, gamma1, beta1, W2, b2, gamma2, beta2):
    new_xyz = _fps_pallas(xyz)
    idx = _knn_pallas(xyz, new_xyz)
    grouped_xyz = _gather_rows(xyz, idx) - new_xyz[:, :, None, :]
    grouped_points = jnp.concatenate([grouped_xyz, _gather_rows(points, idx)], axis=-1)
    new_points = _mlp_bn_max_pallas(grouped_points,
                                    [(W0, b0, gamma0, beta0), (W1, b1, gamma1, beta1), (W2, b2, gamma2, beta2)])
    return new_xyz, new_points
